# baseline (speedup 1.0000x reference)
import jax
import jax.numpy as jnp
from jax.experimental import pallas as pl
from jax.experimental.pallas import tpu as pltpu

_HEADS = 16
_NUM_BUCKETS = 32
_SCALE = 0.125
_THRESHOLDS = (19, 23, 42, 218)
_MAX_BUCKET = 19

_BI = 256
_SEQ = 2048
_WW = _BI + _SEQ


def _bias_add_kernel(table_ref, qk_ref, out_ref):
    ib = pl.program_id(1)
    i0 = ib * _BI

    sub = jax.lax.broadcasted_iota(jnp.int32, (8, _WW), 0)
    lane = jax.lax.broadcasted_iota(jnp.int32, (8, _WW), 1)
    n = jnp.maximum(i0 + (_BI - 8) + sub - lane, 0)

    large = 15 + sum((n >= t).astype(jnp.int32) for t in _THRESHOLDS)
    bucket = jnp.where(n < 16, n, jnp.where(n == 16, 0, large))

    w = jnp.full((8, _WW), table_ref[0, 0, 0] * _SCALE, jnp.float32)
    for b in range(1, _MAX_BUCKET + 1):
        w = jnp.where(bucket == b, table_ref[0, 0, b] * _SCALE, w)

    for g in range(_BI // 8):
        off = (_BI - 8) - 8 * g
        r = 8 * g
        out_ref[0, 0, r:r + 8, :] = (
            qk_ref[0, 0, r:r + 8, :] + w[:, off:off + _SEQ])


def kernel(qk_dots, rel_bias_table):
    n_ib = _SEQ // _BI
    table_t = jnp.transpose(rel_bias_table).reshape(_HEADS, 1, _NUM_BUCKETS)
    return pl.pallas_call(
        _bias_add_kernel,
        grid=(_HEADS, n_ib),
        in_specs=[
            pl.BlockSpec((1, 1, _NUM_BUCKETS), lambda h, ib: (h, 0, 0)),
            pl.BlockSpec((1, 1, _BI, _SEQ), lambda h, ib: (0, h, ib, 0)),
        ],
        out_specs=pl.BlockSpec((1, 1, _BI, _SEQ), lambda h, ib: (0, h, ib, 0)),
        out_shape=jax.ShapeDtypeStruct(qk_dots.shape, qk_dots.dtype),
        compiler_params=pltpu.CompilerParams(
            dimension_semantics=("parallel", "parallel")),
    )(table_t, qk_dots)


# speedup vs baseline: 1.0425x; 1.0425x over previous
import jax
import jax.numpy as jnp
from jax.experimental import pallas as pl
from jax.experimental.pallas import tpu as pltpu

_HEADS = 16
_NUM_BUCKETS = 32
_SCALE = 0.125
_THRESHOLDS = (19, 23, 42, 218)
_MAX_BUCKET = 19

_BI = 256
_SEQ = 2048
_WW = _BI + _SEQ
_LW = 4096


def _bias_add_kernel(table_ref, qk_ref, out_ref, lsw_ref):
    ib = pl.program_id(1)
    n_ib = pl.num_programs(1)

    @pl.when(ib == 0)
    def _():
        sub = jax.lax.broadcasted_iota(jnp.int32, (8, _LW), 0)
        lane = jax.lax.broadcasted_iota(jnp.int32, (8, _LW), 1)
        n = jnp.maximum((_SEQ - _BI) + (_BI - 8) + sub - lane, 0)
        large = 15 + sum((n >= t).astype(jnp.int32) for t in _THRESHOLDS)
        bucket = jnp.where(n < 16, n, jnp.where(n == 16, 0, large))
        w = jnp.full((8, _LW), table_ref[0, 0, 0] * _SCALE, jnp.float32)
        for b in range(1, _MAX_BUCKET + 1):
            w = jnp.where(bucket == b, table_ref[0, 0, b] * _SCALE, w)
        lsw_ref[...] = w

    w = lsw_ref[:, pl.ds(_BI * (n_ib - 1 - ib), _WW)]

    for g in range(_BI // 8):
        off = (_BI - 8) - 8 * g
        r = 8 * g
        out_ref[0, 0, r:r + 8, :] = (
            qk_ref[0, 0, r:r + 8, :] + w[:, off:off + _SEQ])


def kernel(qk_dots, rel_bias_table):
    n_ib = _SEQ // _BI
    table_t = jnp.transpose(rel_bias_table).reshape(_HEADS, 1, _NUM_BUCKETS)
    return pl.pallas_call(
        _bias_add_kernel,
        grid=(_HEADS, n_ib),
        in_specs=[
            pl.BlockSpec((1, 1, _NUM_BUCKETS), lambda h, ib: (h, 0, 0)),
            pl.BlockSpec((1, 1, _BI, _SEQ), lambda h, ib: (0, h, ib, 0)),
        ],
        out_specs=pl.BlockSpec((1, 1, _BI, _SEQ), lambda h, ib: (0, h, ib, 0)),
        out_shape=jax.ShapeDtypeStruct(qk_dots.shape, qk_dots.dtype),
        scratch_shapes=[pltpu.VMEM((8, _LW), jnp.float32)],
        compiler_params=pltpu.CompilerParams(
            dimension_semantics=("parallel", "arbitrary")),
    )(table_t, qk_dots)
